# baseline (speedup 1.0000x reference)
import functools

import jax
import jax.numpy as jnp
from jax import lax
from jax.experimental import pallas as pl
from jax.experimental.pallas import tpu as pltpu
from jax.experimental.pallas import tpu_sc as plsc

_BATCH = 4096
_HIST = 200
_HALF = 32
_NROWS = 1000001
_N = _BATCH * _HIST
_NW = 32
_CH = 128
_GK = 2
_NTB = _BATCH // _CH
_ROWS = _HIST * _NTB
_ROWS_W = _ROWS // _NW
_NGRP = _ROWS_W // _GK
_NIT = _NGRP // 2


_TBLK = 512
_TNB = (_NROWS + _TBLK - 1) // _TBLK


def _relayout_body(a_ref, b_ref, oa_ref, ob_ref):
    oa_ref[...] = a_ref[...].T
    ob_ref[...] = b_ref[...].T


_relayout = pl.pallas_call(
    _relayout_body,
    grid=(_TNB,),
    in_specs=[pl.BlockSpec((_HALF, _TBLK), lambda i: (0, i))] * 2,
    out_specs=[pl.BlockSpec((_TBLK, _HALF), lambda i: (i, 0))] * 2,
    out_shape=[jax.ShapeDtypeStruct((_NROWS, _HALF), jnp.float32)] * 2,
)


def _scratch_set():
    return [
        pltpu.VMEM((_GK, _CH), jnp.int32),
        pltpu.VMEM((_GK, _CH), jnp.int32),
        pltpu.VMEM((_GK, _CH, _HALF), jnp.float32),
        pltpu.VMEM((_GK, _CH, _HALF), jnp.float32),
        pltpu.VMEM((_GK, 8, 8, _CH + 1), jnp.float32),
        pltpu.SemaphoreType.DMA,
        pltpu.SemaphoreType.DMA,
        pltpu.SemaphoreType.DMA,
        pltpu.SemaphoreType.DMA,
        pltpu.SemaphoreType.DMA,
    ]


@functools.partial(
    pl.kernel,
    out_type=jax.ShapeDtypeStruct((_HIST, 8, _NTB, 8, _CH), jnp.float32),
    mesh=plsc.VectorSubcoreMesh(core_axis_name="c", subcore_axis_name="s"),
    scratch_types=_scratch_set() + _scratch_set(),
    compiler_params=pltpu.CompilerParams(
        use_tc_tiling_on_sc=False, needs_layout_passes=False),
)
def _gather_kernel(ids3, item_t, year_t, ylut, out,
                   idx0, yidx0, ie0, ye0, tv0, sa0, sb0, sc0, sd0, se0,
                   idx1, yidx1, ie1, ye1, tv1, sa1, sb1, sc1, sd1, se1):
    wid = lax.axis_index("s") * 2 + lax.axis_index("c")
    wrow = wid * _ROWS_W

    sets = (
        (idx0, yidx0, ie0, ye0, tv0, sa0, sb0, sc0, sd0, se0),
        (idx1, yidx1, ie1, ye1, tv1, sa1, sb1, sc1, sd1, se1),
    )

    lane = lax.iota(jnp.int32, 16)
    eiv = lax.rem(lane, 8)
    tevs = [lane // 8 + t for t in range(8)]

    def out_desc(row, bufs):
        idx, yidx, ie, ye, tv, sa, sb, sc, sd, se = bufs
        h = row // _NTB
        tb0 = lax.rem(row, _NTB)
        descs = []
        for c in range(_GK):
            descs.append(pltpu.make_async_copy(
                tv.at[c, pl.ds(0, 8), pl.ds(0, 8), pl.ds(0, _CH)],
                out.at[h, pl.ds(0, 8), tb0 + c], se))
        return descs

    def transpose_rows(src, tv, c, te0):
        tev_lo, tev_hi = tevs[te0], tevs[te0 + 2]

        def rbody(r, carry):
            for u in range(4):
                rr = r * 4 + u
                bv = jnp.full((16,), rr, jnp.int32)
                v0 = src[c, rr, pl.ds(0, 16)]
                v1 = src[c, rr, pl.ds(16, 16)]
                plsc.store_scatter(tv.at[c], [tev_lo, eiv, bv], v0)
                plsc.store_scatter(tv.at[c], [tev_hi, eiv, bv], v1)
            return carry

        lax.fori_loop(0, _CH // 4, rbody, 0)

    def run_group(row, bufs):
        idx, yidx, ie, ye, tv, sa, sb, sc, sd, se = bufs
        h = row // _NTB
        tb0 = lax.rem(row, _NTB)
        pltpu.async_copy(ids3.at[h, pl.ds(tb0, _GK)], idx, sa).wait()
        for c in range(_GK):
            pltpu.async_copy(ylut.at[idx.at[c]], yidx.at[c], sb)
            pltpu.async_copy(item_t.at[idx.at[c]], ie.at[c], sc)
        for c in range(_GK):
            pltpu.make_async_copy(ylut.at[idx.at[c]], yidx.at[c], sb).wait()
            pltpu.async_copy(year_t.at[yidx.at[c]], ye.at[c], sd)
        for c in range(_GK):
            pltpu.make_async_copy(item_t.at[idx.at[c]], ie.at[c], sc).wait()
            transpose_rows(ie, tv, c, 0)
        for c in range(_GK):
            pltpu.make_async_copy(year_t.at[yidx.at[c]], ye.at[c], sd).wait()
            transpose_rows(ye, tv, c, 4)
        for d in out_desc(row, bufs):
            d.start()

    def step(i, carry):
        for s in (0, 1):
            row = wrow + (2 * i + s) * _GK

            @pl.when(i > 0)
            def _():
                for d in out_desc(row - 2 * _GK, sets[s]):
                    d.wait()

            run_group(row, sets[s])
        return carry

    lax.fori_loop(0, _NIT, step, 0)

    for s in (0, 1):
        for d in out_desc(wrow + (2 * (_NIT - 1) + s) * _GK, sets[s]):
            d.wait()


def kernel(item_ids, item_emb, year_emb, year_lookup_table):
    ids3 = item_ids.T.reshape(_HIST, _NTB, _CH)
    item_rm, year_rm = _relayout(item_emb.T, year_emb.T)
    out = _gather_kernel(ids3, item_rm, year_rm, year_lookup_table)
    return out.transpose(2, 4, 0, 1, 3).reshape(_BATCH, _HIST, 2 * _HALF)


# speedup vs baseline: 1.3790x; 1.3790x over previous
import functools

import jax
import jax.numpy as jnp
from jax import lax
from jax.experimental import pallas as pl
from jax.experimental.pallas import tpu as pltpu
from jax.experimental.pallas import tpu_sc as plsc

_BATCH = 4096
_HIST = 200
_HALF = 32
_NROWS = 1000001
_N = _BATCH * _HIST
_NW = 32
_CH = 128
_GK = 2
_NTB = _BATCH // _CH
_ROWS = _HIST * _NTB
_ROWS_W = _ROWS // _NW
_NGRP = _ROWS_W // _GK
_NIT = _NGRP // 2


_NCT = (_NROWS + _CH - 1) // _CH
_CROWS = _NCT * _CH
_PB = 64
_PNB = (_NCT + _PB - 1) // _PB


def _permute_body(a_ref, b_ref, oa_ref, ob_ref):
    for src, dst in ((a_ref, oa_ref), (b_ref, ob_ref)):
        for j in range(_PB):
            dst[:, j] = src[:, pl.ds(j * _CH, _CH)].reshape(4, 8, _CH)


_permute = pl.pallas_call(
    _permute_body,
    grid=(_PNB,),
    in_specs=[pl.BlockSpec((_HALF, _PB * _CH), lambda i: (0, i))] * 2,
    out_specs=[pl.BlockSpec((4, _PB, 8, _CH), lambda i: (0, i, 0, 0))] * 2,
    out_shape=[jax.ShapeDtypeStruct((4, _NCT, 8, _CH), jnp.float32)] * 2,
)


_UPW = (_NCT + 15) // 16


def _rl_scratch():
    return [
        pltpu.VMEM((4, 1, 8, _CH), jnp.float32),
        pltpu.VMEM((_CH, _HALF + 1), jnp.float32),
        pltpu.SemaphoreType.DMA,
        pltpu.SemaphoreType.DMA,
    ]


@functools.partial(
    pl.kernel,
    out_type=[jax.ShapeDtypeStruct((_CROWS, _HALF), jnp.float32)] * 2,
    mesh=plsc.VectorSubcoreMesh(core_axis_name="c", subcore_axis_name="s"),
    scratch_types=_rl_scratch() + _rl_scratch(),
    compiler_params=pltpu.CompilerParams(
        use_tc_tiling_on_sc=False, needs_layout_passes=False),
)
def _relayout_kernel(t3a, t3b, oa, ob,
                     vin0, vt0, si0, so0, vin1, vt1, si1, so1):
    wid = lax.axis_index("s") * 2 + lax.axis_index("c")
    widt = lax.rem(wid, 16)

    lane = lax.iota(jnp.int32, 16)
    colvs = [jnp.full((16,), e, jnp.int32) for e in range(_HALF)]
    sets = ((vin0, vt0, si0, so0), (vin1, vt1, si1, so1))

    def in_desc(t3, u, bufs):
        vin, vt, si, so = bufs
        return pltpu.make_async_copy(t3.at[pl.ds(0, 4), pl.ds(u, 1)], vin, si)

    def out_desc(out, u, bufs):
        vin, vt, si, so = bufs
        return pltpu.make_async_copy(
            vt.at[pl.ds(0, _CH), pl.ds(0, _HALF)],
            out.at[pl.ds(u * _CH, _CH), pl.ds(0, _HALF)], so)

    def run_table(t3, out):
        def step(i, carry):
            for s in (0, 1):
                u = (2 * i + s) * 16 + widt
                bufs = sets[s]
                vin, vt, si, so = bufs

                @pl.when(u < _NCT)
                def _():
                    @pl.when(i > 0)
                    def _():
                        uprev = (2 * (i - 1) + s) * 16 + widt
                        out_desc(out, uprev, bufs).wait()

                    in_desc(t3, u, bufs).start()
                    in_desc(t3, u, bufs).wait()

                    def gbody(g, c2):
                        rowv = lane + g * 16
                        for e in range(_HALF):
                            v = vin[e // 8, 0, e % 8, pl.ds(g * 16, 16)]
                            plsc.store_scatter(vt, [rowv, colvs[e]], v)
                        return c2

                    lax.fori_loop(0, _CH // 16, gbody, 0)
                    out_desc(out, u, bufs).start()
            return carry

        lax.fori_loop(0, (_UPW + 1) // 2, step, 0)
        for s in (0, 1):
            out_desc(out, 0, sets[s]).wait()

    @pl.when(wid < 16)
    def _():
        run_table(t3a, oa)

    @pl.when(wid >= 16)
    def _():
        run_table(t3b, ob)


def _scratch_set():
    return [
        pltpu.VMEM((_GK, _CH), jnp.int32),
        pltpu.VMEM((_GK, _CH), jnp.int32),
        pltpu.VMEM((_GK, _CH, _HALF), jnp.float32),
        pltpu.VMEM((_GK, _CH, _HALF), jnp.float32),
        pltpu.VMEM((_GK, 8, 8, _CH + 1), jnp.float32),
        pltpu.SemaphoreType.DMA,
        pltpu.SemaphoreType.DMA,
        pltpu.SemaphoreType.DMA,
        pltpu.SemaphoreType.DMA,
        pltpu.SemaphoreType.DMA,
    ]


@functools.partial(
    pl.kernel,
    out_type=jax.ShapeDtypeStruct((_HIST, 8, _NTB, 8, _CH), jnp.float32),
    mesh=plsc.VectorSubcoreMesh(core_axis_name="c", subcore_axis_name="s"),
    scratch_types=_scratch_set() + _scratch_set(),
    compiler_params=pltpu.CompilerParams(
        use_tc_tiling_on_sc=False, needs_layout_passes=False),
)
def _gather_kernel(ids3, item_t, year_t, ylut, out,
                   idx0, yidx0, ie0, ye0, tv0, sa0, sb0, sc0, sd0, se0,
                   idx1, yidx1, ie1, ye1, tv1, sa1, sb1, sc1, sd1, se1):
    wid = lax.axis_index("s") * 2 + lax.axis_index("c")
    wrow = wid * _ROWS_W

    sets = (
        (idx0, yidx0, ie0, ye0, tv0, sa0, sb0, sc0, sd0, se0),
        (idx1, yidx1, ie1, ye1, tv1, sa1, sb1, sc1, sd1, se1),
    )

    lane = lax.iota(jnp.int32, 16)
    eiv = lax.rem(lane, 8)
    tevs = [lane // 8 + t for t in range(8)]

    def out_desc(row, bufs):
        idx, yidx, ie, ye, tv, sa, sb, sc, sd, se = bufs
        h = row // _NTB
        tb0 = lax.rem(row, _NTB)
        descs = []
        for c in range(_GK):
            descs.append(pltpu.make_async_copy(
                tv.at[c, pl.ds(0, 8), pl.ds(0, 8), pl.ds(0, _CH)],
                out.at[h, pl.ds(0, 8), tb0 + c], se))
        return descs

    def transpose_rows(src, tv, c, te0):
        tev_lo, tev_hi = tevs[te0], tevs[te0 + 2]

        def rbody(r, carry):
            for u in range(4):
                rr = r * 4 + u
                bv = jnp.full((16,), rr, jnp.int32)
                v0 = src[c, rr, pl.ds(0, 16)]
                v1 = src[c, rr, pl.ds(16, 16)]
                plsc.store_scatter(tv.at[c], [tev_lo, eiv, bv], v0)
                plsc.store_scatter(tv.at[c], [tev_hi, eiv, bv], v1)
            return carry

        lax.fori_loop(0, _CH // 4, rbody, 0)

    def run_group(row, bufs):
        idx, yidx, ie, ye, tv, sa, sb, sc, sd, se = bufs
        h = row // _NTB
        tb0 = lax.rem(row, _NTB)
        pltpu.async_copy(ids3.at[h, pl.ds(tb0, _GK)], idx, sa).wait()
        for c in range(_GK):
            pltpu.async_copy(ylut.at[idx.at[c]], yidx.at[c], sb)
            pltpu.async_copy(item_t.at[idx.at[c]], ie.at[c], sc)
        for c in range(_GK):
            pltpu.make_async_copy(ylut.at[idx.at[c]], yidx.at[c], sb).wait()
            pltpu.async_copy(year_t.at[yidx.at[c]], ye.at[c], sd)
        for c in range(_GK):
            pltpu.make_async_copy(item_t.at[idx.at[c]], ie.at[c], sc).wait()
            transpose_rows(ie, tv, c, 0)
        for c in range(_GK):
            pltpu.make_async_copy(year_t.at[yidx.at[c]], ye.at[c], sd).wait()
            transpose_rows(ye, tv, c, 4)
        for d in out_desc(row, bufs):
            d.start()

    def step(i, carry):
        for s in (0, 1):
            row = wrow + (2 * i + s) * _GK

            @pl.when(i > 0)
            def _():
                for d in out_desc(row - 2 * _GK, sets[s]):
                    d.wait()

            run_group(row, sets[s])
        return carry

    lax.fori_loop(0, _NIT, step, 0)

    for s in (0, 1):
        for d in out_desc(wrow + (2 * (_NIT - 1) + s) * _GK, sets[s]):
            d.wait()


def kernel(item_ids, item_emb, year_emb, year_lookup_table):
    ids3 = item_ids.T.reshape(_HIST, _NTB, _CH)
    t3a, t3b = _permute(item_emb.T, year_emb.T)
    item_rm, year_rm = _relayout_kernel(t3a, t3b)
    out = _gather_kernel(ids3, item_rm, year_rm, year_lookup_table)
    return out.transpose(2, 4, 0, 1, 3).reshape(_BATCH, _HIST, 2 * _HALF)


# speedup vs baseline: 1.8872x; 1.3685x over previous
import functools

import jax
import jax.numpy as jnp
from jax import lax
from jax.experimental import pallas as pl
from jax.experimental.pallas import tpu as pltpu
from jax.experimental.pallas import tpu_sc as plsc

_BATCH = 4096
_HIST = 200
_HALF = 32
_NROWS = 1000001
_N = _BATCH * _HIST
_NW = 32
_CH = 128
_GK = 2
_NTB = _BATCH // _CH
_ROWS = _HIST * _NTB
_ROWS_W = _ROWS // _NW
_NGRP = _ROWS_W // _GK
_NIT = _NGRP // 2


_NCT = (_NROWS + _CH - 1) // _CH
_CROWS = _NCT * _CH
_PB = 64
_PNB = (_NCT + _PB - 1) // _PB


def _permute_body(a_ref, b_ref, oa_ref, ob_ref):
    for src, dst in ((a_ref, oa_ref), (b_ref, ob_ref)):
        for j in range(_PB):
            dst[:, j] = src[:, pl.ds(j * _CH, _CH)].reshape(4, 8, _CH)


_permute = pl.pallas_call(
    _permute_body,
    grid=(_PNB,),
    in_specs=[pl.BlockSpec((_HALF, _PB * _CH), lambda i: (0, i))] * 2,
    out_specs=[pl.BlockSpec((4, _PB, 8, _CH), lambda i: (0, i, 0, 0))] * 2,
    out_shape=[jax.ShapeDtypeStruct((4, _NCT, 8, _CH), jnp.float32)] * 2,
)


_UPW = (_NCT + 15) // 16


def _rl_scratch():
    return [
        pltpu.VMEM((4, 1, 8, _CH), jnp.float32),
        pltpu.VMEM((_CH, _HALF + 1), jnp.float32),
        pltpu.SemaphoreType.DMA,
        pltpu.SemaphoreType.DMA,
    ]


@functools.partial(
    pl.kernel,
    out_type=[jax.ShapeDtypeStruct((_CROWS, _HALF), jnp.float32)] * 2,
    mesh=plsc.VectorSubcoreMesh(core_axis_name="c", subcore_axis_name="s"),
    scratch_types=_rl_scratch() + _rl_scratch(),
    compiler_params=pltpu.CompilerParams(
        use_tc_tiling_on_sc=False, needs_layout_passes=False),
)
def _relayout_kernel(t3a, t3b, oa, ob,
                     vin0, vt0, si0, so0, vin1, vt1, si1, so1):
    wid = lax.axis_index("s") * 2 + lax.axis_index("c")
    widt = lax.rem(wid, 16)

    lane = lax.iota(jnp.int32, 16)
    colvs = [jnp.full((16,), e, jnp.int32) for e in range(_HALF)]
    sets = ((vin0, vt0, si0, so0), (vin1, vt1, si1, so1))

    def in_desc(t3, u, bufs):
        vin, vt, si, so = bufs
        return pltpu.make_async_copy(t3.at[pl.ds(0, 4), pl.ds(u, 1)], vin, si)

    def out_desc(out, u, bufs):
        vin, vt, si, so = bufs
        return pltpu.make_async_copy(
            vt.at[pl.ds(0, _CH), pl.ds(0, _HALF)],
            out.at[pl.ds(u * _CH, _CH), pl.ds(0, _HALF)], so)

    def run_table(t3, out):
        for s in (0, 1):
            @pl.when(s * 16 + widt < _NCT)
            def _():
                in_desc(t3, s * 16 + widt, sets[s]).start()

        def step(i, carry):
            for s in (0, 1):
                u = (2 * i + s) * 16 + widt
                bufs = sets[s]
                vin, vt, si, so = bufs

                @pl.when(u < _NCT)
                def _():
                    @pl.when(i > 0)
                    def _():
                        uprev = (2 * (i - 1) + s) * 16 + widt
                        out_desc(out, uprev, bufs).wait()

                    in_desc(t3, u, bufs).wait()

                    def gbody(g, c2):
                        rowv = lane + g * 16
                        for e in range(_HALF):
                            v = vin[e // 8, 0, e % 8, pl.ds(g * 16, 16)]
                            plsc.store_scatter(vt, [rowv, colvs[e]], v)
                        return c2

                    lax.fori_loop(0, _CH // 16, gbody, 0)
                    out_desc(out, u, bufs).start()

                    unext = (2 * (i + 1) + s) * 16 + widt

                    @pl.when(unext < _NCT)
                    def _():
                        in_desc(t3, unext, bufs).start()
            return carry

        lax.fori_loop(0, (_UPW + 1) // 2, step, 0)
        for s in (0, 1):
            out_desc(out, 0, sets[s]).wait()

    @pl.when(wid < 16)
    def _():
        run_table(t3a, oa)

    @pl.when(wid >= 16)
    def _():
        run_table(t3b, ob)


def _scratch_set():
    return [
        pltpu.VMEM((_GK, _CH), jnp.int32),
        pltpu.VMEM((_GK, _CH), jnp.int32),
        pltpu.VMEM((_GK, _CH, _HALF), jnp.float32),
        pltpu.VMEM((_GK, _CH, _HALF), jnp.float32),
        pltpu.VMEM((_GK, 8, 8, _CH + 1), jnp.float32),
        pltpu.SemaphoreType.DMA,
        pltpu.SemaphoreType.DMA,
        pltpu.SemaphoreType.DMA,
        pltpu.SemaphoreType.DMA,
        pltpu.SemaphoreType.DMA,
    ]


@functools.partial(
    pl.kernel,
    out_type=jax.ShapeDtypeStruct((_HIST, 8, _NTB, 8, _CH), jnp.float32),
    mesh=plsc.VectorSubcoreMesh(core_axis_name="c", subcore_axis_name="s"),
    scratch_types=_scratch_set() + _scratch_set(),
    compiler_params=pltpu.CompilerParams(
        use_tc_tiling_on_sc=False, needs_layout_passes=False),
)
def _gather_kernel(ids3, item_t, year_t, ylut, out,
                   idx0, yidx0, ie0, ye0, tv0, sa0, sb0, sc0, sd0, se0,
                   idx1, yidx1, ie1, ye1, tv1, sa1, sb1, sc1, sd1, se1):
    wid = lax.axis_index("s") * 2 + lax.axis_index("c")
    wrow = wid * _ROWS_W

    sets = (
        (idx0, yidx0, ie0, ye0, tv0, sa0, sb0, sc0, sd0, se0),
        (idx1, yidx1, ie1, ye1, tv1, sa1, sb1, sc1, sd1, se1),
    )

    lane = lax.iota(jnp.int32, 16)
    eiv = lax.rem(lane, 8)
    tevs = [lane // 8 + t for t in range(8)]

    def out_desc(row, bufs):
        idx, yidx, ie, ye, tv, sa, sb, sc, sd, se = bufs
        h = row // _NTB
        tb0 = lax.rem(row, _NTB)
        descs = []
        for c in range(_GK):
            descs.append(pltpu.make_async_copy(
                tv.at[c, pl.ds(0, 8), pl.ds(0, 8), pl.ds(0, _CH)],
                out.at[h, pl.ds(0, 8), tb0 + c], se))
        return descs

    def transpose_rows(src, tv, c, te0):
        tev_lo, tev_hi = tevs[te0], tevs[te0 + 2]

        def rbody(r, carry):
            for u in range(4):
                rr = r * 4 + u
                bv = jnp.full((16,), rr, jnp.int32)
                v0 = src[c, rr, pl.ds(0, 16)]
                v1 = src[c, rr, pl.ds(16, 16)]
                plsc.store_scatter(tv.at[c], [tev_lo, eiv, bv], v0)
                plsc.store_scatter(tv.at[c], [tev_hi, eiv, bv], v1)
            return carry

        lax.fori_loop(0, _CH // 4, rbody, 0)

    def a_desc(row, bufs):
        idx = bufs[0]
        h = row // _NTB
        tb0 = lax.rem(row, _NTB)
        return pltpu.make_async_copy(ids3.at[h, pl.ds(tb0, _GK)], idx,
                                     bufs[5])

    for s in (0, 1):
        a_desc(wrow + s * _GK, sets[s]).start()

    def step(i, carry):
        rows = [wrow + (2 * i + s) * _GK for s in (0, 1)]
        for s in (0, 1):
            idx, yidx, ie, ye, tv, sa, sb, sc, sd, se = sets[s]
            a_desc(rows[s], sets[s]).wait()
            for c in range(_GK):
                pltpu.async_copy(ylut.at[idx.at[c]], yidx.at[c], sb)
                pltpu.async_copy(item_t.at[idx.at[c]], ie.at[c], sc)
        for s in (0, 1):
            idx, yidx, ie, ye, tv, sa, sb, sc, sd, se = sets[s]
            for c in range(_GK):
                pltpu.make_async_copy(ylut.at[idx.at[c]], yidx.at[c],
                                      sb).wait()
                pltpu.async_copy(year_t.at[yidx.at[c]], ye.at[c], sd)
        for s in (0, 1):
            idx, yidx, ie, ye, tv, sa, sb, sc, sd, se = sets[s]

            @pl.when(i > 0)
            def _():
                for d in out_desc(rows[s] - 2 * _GK, sets[s]):
                    d.wait()

            for c in range(_GK):
                pltpu.make_async_copy(item_t.at[idx.at[c]], ie.at[c],
                                      sc).wait()
                transpose_rows(ie, tv, c, 0)
            for c in range(_GK):
                pltpu.make_async_copy(year_t.at[yidx.at[c]], ye.at[c],
                                      sd).wait()
                transpose_rows(ye, tv, c, 4)
            for d in out_desc(rows[s], sets[s]):
                d.start()

            @pl.when(i < _NIT - 1)
            def _():
                a_desc(rows[s] + 2 * _GK, sets[s]).start()
        return carry

    lax.fori_loop(0, _NIT, step, 0)

    for s in (0, 1):
        for d in out_desc(wrow + (2 * (_NIT - 1) + s) * _GK, sets[s]):
            d.wait()


def kernel(item_ids, item_emb, year_emb, year_lookup_table):
    ids3 = item_ids.T.reshape(_HIST, _NTB, _CH)
    t3a, t3b = _permute(item_emb.T, year_emb.T)
    item_rm, year_rm = _relayout_kernel(t3a, t3b)
    out = _gather_kernel(ids3, item_rm, year_rm, year_lookup_table)
    return out.transpose(2, 4, 0, 1, 3).reshape(_BATCH, _HIST, 2 * _HALF)


# speedup vs baseline: 1.9096x; 1.0119x over previous
import functools

import jax
import jax.numpy as jnp
from jax import lax
from jax.experimental import pallas as pl
from jax.experimental.pallas import tpu as pltpu
from jax.experimental.pallas import tpu_sc as plsc

_BATCH = 4096
_HIST = 200
_HALF = 32
_NROWS = 1000001
_N = _BATCH * _HIST
_NW = 32
_CH = 128
_GK = 2
_NTB = _BATCH // _CH
_ROWS = _HIST * _NTB
_ROWS_W = _ROWS // _NW
_NGRP = _ROWS_W // _GK
_NIT = _NGRP // 2


_NCT = (_NROWS + _CH - 1) // _CH
_CROWS = _NCT * _CH
_PB = 128
_PNB = (_NCT + _PB - 1) // _PB


def _permute_body(a_ref, b_ref, oa_ref, ob_ref):
    for src, dst in ((a_ref, oa_ref), (b_ref, ob_ref)):
        for j in range(_PB):
            dst[:, j] = src[:, pl.ds(j * _CH, _CH)].reshape(4, 8, _CH)


_permute = pl.pallas_call(
    _permute_body,
    grid=(_PNB,),
    in_specs=[pl.BlockSpec((_HALF, _PB * _CH), lambda i: (0, i))] * 2,
    out_specs=[pl.BlockSpec((4, _PB, 8, _CH), lambda i: (0, i, 0, 0))] * 2,
    out_shape=[jax.ShapeDtypeStruct((4, _NCT, 8, _CH), jnp.float32)] * 2,
)


_UPW = (_NCT + 15) // 16


def _rl_scratch():
    return [
        pltpu.VMEM((4, 1, 8, _CH), jnp.float32),
        pltpu.VMEM((_CH, _HALF + 1), jnp.float32),
        pltpu.SemaphoreType.DMA,
        pltpu.SemaphoreType.DMA,
    ]


@functools.partial(
    pl.kernel,
    out_type=[jax.ShapeDtypeStruct((_CROWS, _HALF), jnp.float32)] * 2,
    mesh=plsc.VectorSubcoreMesh(core_axis_name="c", subcore_axis_name="s"),
    scratch_types=_rl_scratch() + _rl_scratch(),
    compiler_params=pltpu.CompilerParams(
        use_tc_tiling_on_sc=False, needs_layout_passes=False),
)
def _relayout_kernel(t3a, t3b, oa, ob,
                     vin0, vt0, si0, so0, vin1, vt1, si1, so1):
    wid = lax.axis_index("s") * 2 + lax.axis_index("c")
    widt = lax.rem(wid, 16)

    lane = lax.iota(jnp.int32, 16)
    colvs = [jnp.full((16,), e, jnp.int32) for e in range(_HALF)]
    sets = ((vin0, vt0, si0, so0), (vin1, vt1, si1, so1))

    def in_desc(t3, u, bufs):
        vin, vt, si, so = bufs
        return pltpu.make_async_copy(t3.at[pl.ds(0, 4), pl.ds(u, 1)], vin, si)

    def out_desc(out, u, bufs):
        vin, vt, si, so = bufs
        return pltpu.make_async_copy(
            vt.at[pl.ds(0, _CH), pl.ds(0, _HALF)],
            out.at[pl.ds(u * _CH, _CH), pl.ds(0, _HALF)], so)

    def run_table(t3, out):
        for s in (0, 1):
            @pl.when(s * 16 + widt < _NCT)
            def _():
                in_desc(t3, s * 16 + widt, sets[s]).start()

        def step(i, carry):
            for s in (0, 1):
                u = (2 * i + s) * 16 + widt
                bufs = sets[s]
                vin, vt, si, so = bufs

                @pl.when(u < _NCT)
                def _():
                    @pl.when(i > 0)
                    def _():
                        uprev = (2 * (i - 1) + s) * 16 + widt
                        out_desc(out, uprev, bufs).wait()

                    in_desc(t3, u, bufs).wait()

                    def gbody(g, c2):
                        rowv = lane + g * 16
                        for e in range(_HALF):
                            v = vin[e // 8, 0, e % 8, pl.ds(g * 16, 16)]
                            plsc.store_scatter(vt, [rowv, colvs[e]], v)
                        return c2

                    lax.fori_loop(0, _CH // 16, gbody, 0)
                    out_desc(out, u, bufs).start()

                    unext = (2 * (i + 1) + s) * 16 + widt

                    @pl.when(unext < _NCT)
                    def _():
                        in_desc(t3, unext, bufs).start()
            return carry

        lax.fori_loop(0, (_UPW + 1) // 2, step, 0)
        for s in (0, 1):
            out_desc(out, 0, sets[s]).wait()

    @pl.when(wid < 16)
    def _():
        run_table(t3a, oa)

    @pl.when(wid >= 16)
    def _():
        run_table(t3b, ob)


def _scratch_set():
    return [
        pltpu.VMEM((_GK, _CH), jnp.int32),
        pltpu.VMEM((_GK, _CH), jnp.int32),
        pltpu.VMEM((_GK, _CH, _HALF), jnp.float32),
        pltpu.VMEM((_GK, _CH, _HALF), jnp.float32),
        pltpu.VMEM((_GK, 8, 8, _CH + 1), jnp.float32),
        pltpu.SemaphoreType.DMA,
        pltpu.SemaphoreType.DMA,
        pltpu.SemaphoreType.DMA,
        pltpu.SemaphoreType.DMA,
        pltpu.SemaphoreType.DMA,
    ]


@functools.partial(
    pl.kernel,
    out_type=jax.ShapeDtypeStruct((_HIST, 8, _NTB, 8, _CH), jnp.float32),
    mesh=plsc.VectorSubcoreMesh(core_axis_name="c", subcore_axis_name="s"),
    scratch_types=_scratch_set() + _scratch_set(),
    compiler_params=pltpu.CompilerParams(
        use_tc_tiling_on_sc=False, needs_layout_passes=False),
)
def _gather_kernel(ids3, item_t, year_t, ylut, out,
                   idx0, yidx0, ie0, ye0, tv0, sa0, sb0, sc0, sd0, se0,
                   idx1, yidx1, ie1, ye1, tv1, sa1, sb1, sc1, sd1, se1):
    wid = lax.axis_index("s") * 2 + lax.axis_index("c")
    wrow = wid * _ROWS_W

    sets = (
        (idx0, yidx0, ie0, ye0, tv0, sa0, sb0, sc0, sd0, se0),
        (idx1, yidx1, ie1, ye1, tv1, sa1, sb1, sc1, sd1, se1),
    )

    lane = lax.iota(jnp.int32, 16)
    eiv = lax.rem(lane, 8)
    tevs = [lane // 8 + t for t in range(8)]

    def out_desc(row, bufs):
        idx, yidx, ie, ye, tv, sa, sb, sc, sd, se = bufs
        h = row // _NTB
        tb0 = lax.rem(row, _NTB)
        descs = []
        for c in range(_GK):
            descs.append(pltpu.make_async_copy(
                tv.at[c, pl.ds(0, 8), pl.ds(0, 8), pl.ds(0, _CH)],
                out.at[h, pl.ds(0, 8), tb0 + c], se))
        return descs

    def transpose_rows(src, tv, c, te0):
        tev_lo, tev_hi = tevs[te0], tevs[te0 + 2]

        def rbody(r, carry):
            for u in range(4):
                rr = r * 4 + u
                bv = jnp.full((16,), rr, jnp.int32)
                v0 = src[c, rr, pl.ds(0, 16)]
                v1 = src[c, rr, pl.ds(16, 16)]
                plsc.store_scatter(tv.at[c], [tev_lo, eiv, bv], v0)
                plsc.store_scatter(tv.at[c], [tev_hi, eiv, bv], v1)
            return carry

        lax.fori_loop(0, _CH // 4, rbody, 0)

    def a_desc(row, bufs):
        idx = bufs[0]
        h = row // _NTB
        tb0 = lax.rem(row, _NTB)
        return pltpu.make_async_copy(ids3.at[h, pl.ds(tb0, _GK)], idx,
                                     bufs[5])

    for s in (0, 1):
        a_desc(wrow + s * _GK, sets[s]).start()

    def step(i, carry):
        rows = [wrow + (2 * i + s) * _GK for s in (0, 1)]
        for s in (0, 1):
            idx, yidx, ie, ye, tv, sa, sb, sc, sd, se = sets[s]
            a_desc(rows[s], sets[s]).wait()
            for c in range(_GK):
                pltpu.async_copy(ylut.at[idx.at[c]], yidx.at[c], sb)
                pltpu.async_copy(item_t.at[idx.at[c]], ie.at[c], sc)
        for s in (0, 1):
            idx, yidx, ie, ye, tv, sa, sb, sc, sd, se = sets[s]
            for c in range(_GK):
                pltpu.make_async_copy(ylut.at[idx.at[c]], yidx.at[c],
                                      sb).wait()
                pltpu.async_copy(year_t.at[yidx.at[c]], ye.at[c], sd)
        for s in (0, 1):
            idx, yidx, ie, ye, tv, sa, sb, sc, sd, se = sets[s]

            @pl.when(i > 0)
            def _():
                for d in out_desc(rows[s] - 2 * _GK, sets[s]):
                    d.wait()

            for c in range(_GK):
                pltpu.make_async_copy(item_t.at[idx.at[c]], ie.at[c],
                                      sc).wait()
                transpose_rows(ie, tv, c, 0)
            for c in range(_GK):
                pltpu.make_async_copy(year_t.at[yidx.at[c]], ye.at[c],
                                      sd).wait()
                transpose_rows(ye, tv, c, 4)
            for d in out_desc(rows[s], sets[s]):
                d.start()

            @pl.when(i < _NIT - 1)
            def _():
                a_desc(rows[s] + 2 * _GK, sets[s]).start()
        return carry

    lax.fori_loop(0, _NIT, step, 0)

    for s in (0, 1):
        for d in out_desc(wrow + (2 * (_NIT - 1) + s) * _GK, sets[s]):
            d.wait()


def kernel(item_ids, item_emb, year_emb, year_lookup_table):
    ids3 = item_ids.T.reshape(_HIST, _NTB, _CH)
    t3a, t3b = _permute(item_emb.T, year_emb.T)
    item_rm, year_rm = _relayout_kernel(t3a, t3b)
    out = _gather_kernel(ids3, item_rm, year_rm, year_lookup_table)
    return out.transpose(2, 4, 0, 1, 3).reshape(_BATCH, _HIST, 2 * _HALF)
